# Initial kernel scaffold; baseline (speedup 1.0000x reference)
import jax
import jax.numpy as jnp
from jax.experimental import pallas as pl


def kernel(x, edge_index, batch_indice, smile_llm, params):
    raise NotImplementedError("write your pallas kernel here")


# speedup vs baseline: 14.9835x; 14.9835x over previous
import functools

import jax
import jax.numpy as jnp
from jax import lax
from jax.experimental import pallas as pl
from jax.experimental.pallas import tpu as pltpu
from jax.experimental.pallas import tpu_sc as plsc

N = 10000
E = 320000
G = 64
D = 128
NC = 2
NS = 16
NW = NC * NS
EP = E // NW
CH = 125
NCHUNK = EP // CH
RB = 1000
NRB = N // RB
RPS = N // NS
EPS = 1e-5

_sc_mesh = plsc.VectorSubcoreMesh(core_axis_name="c", subcore_axis_name="s")


def _deg_body(dst_hbm, degp_hbm, dstv, degv):
    c = lax.axis_index("c")
    s = lax.axis_index("s")
    wid = c * NS + s
    pltpu.sync_copy(dst_hbm.at[wid], dstv)

    def zero(i, carry):
        degv[pl.ds(i * 16, 16)] = jnp.zeros((16,), jnp.float32)
        return carry

    lax.fori_loop(0, N // 16, zero, 0)
    ones = jnp.ones((16,), jnp.float32)

    def acc(i, carry):
        idx = dstv[pl.ds(i * 16, 16)]
        plsc.addupdate_scatter(degv, [idx], ones)
        return carry

    lax.fori_loop(0, EP // 16, acc, 0)
    for jb in range(NRB):
        pltpu.sync_copy(degv.at[pl.ds(jb * RB, RB)], degp_hbm.at[jb, wid])


_deg_kernel = functools.partial(
    pl.kernel,
    out_type=jax.ShapeDtypeStruct((NRB, NW, RB), jnp.float32),
    mesh=_sc_mesh,
    compiler_params=pltpu.CompilerParams(needs_layout_passes=False, use_tc_tiling_on_sc=False),
    scratch_types=[
        pltpu.VMEM((EP,), jnp.int32),
        pltpu.VMEM((N,), jnp.float32),
    ],
)(_deg_body)


def _mp_body(zt_hbm, src_hbm, dst_hbm, acc2_hbm, srcv, dstv, rows, acc_sh,
             gsem, ssem):
    c = lax.axis_index("c")
    s = lax.axis_index("s")
    wid = c * NS + s
    pltpu.sync_copy(src_hbm.at[wid], srcv)
    pltpu.sync_copy(dst_hbm.at[wid], dstv)
    pltpu.sync_copy(zt_hbm.at[pl.ds(s * RPS, RPS)],
                    acc_sh.at[pl.ds(s * RPS, RPS)])
    plsc.subcore_barrier()

    def chunk(j, carry):
        pltpu.async_copy(zt_hbm.at[srcv.at[j]], rows, gsem).wait()
        pltpu.async_copy(rows, acc_sh.at[dstv.at[j]], ssem, add=True).wait()
        return carry

    lax.fori_loop(0, NCHUNK, chunk, 0)
    plsc.subcore_barrier()
    pltpu.sync_copy(acc_sh.at[pl.ds(s * RPS, RPS)],
                    acc2_hbm.at[c, pl.ds(s * RPS, RPS)])


_mp_kernel = functools.partial(
    pl.kernel,
    out_type=jax.ShapeDtypeStruct((NC, N, D), jnp.float32),
    mesh=_sc_mesh,
    compiler_params=pltpu.CompilerParams(needs_layout_passes=False, use_tc_tiling_on_sc=False),
    scratch_types=[
        pltpu.VMEM((NCHUNK, CH), jnp.int32),
        pltpu.VMEM((NCHUNK, CH), jnp.int32),
        pltpu.VMEM((CH, D), jnp.float32),
        pltpu.VMEM_SHARED((N, D), jnp.float32),
        pltpu.SemaphoreType.DMA,
        pltpu.SemaphoreType.DMA,
    ],
)(_mp_body)


def _graph_bias(smile_ref, wproj_ref, bproj_ref, wrep_ref):
    s = jnp.maximum(
        lax.dot_general(smile_ref[...], wproj_ref[...],
                        (((1,), (0,)), ((), ())),
                        preferred_element_type=jnp.float32)
        + bproj_ref[...], 0.0)
    return lax.dot_general(s, wrep_ref[...], (((1,), (0,)), ((), ())),
                           preferred_element_type=jnp.float32)


def _project(h, batch_ref, wh_ref, bias):
    oh = (batch_ref[...] ==
          lax.broadcasted_iota(jnp.int32, (h.shape[0], G), 1)
          ).astype(jnp.float32)
    z = lax.dot_general(h, wh_ref[...], (((1,), (0,)), ((), ())),
                        preferred_element_type=jnp.float32)
    z = z + lax.dot_general(oh, bias, (((1,), (0,)), ((), ())),
                            preferred_element_type=jnp.float32)
    return z


def _layer0_body(degp_ref, x_ref, batch_ref, smile_ref, wproj_ref, bproj_ref,
                 wrep_ref, wh_ref, dinv_ref, zt_ref):
    degsum = lax.dot_general(degp_ref[...].reshape(NW, RB),
                             jnp.ones((NW, 1), jnp.float32),
                             (((0,), (0,)), ((), ())),
                             preferred_element_type=jnp.float32)
    dinv = lax.rsqrt(degsum + 1.0)
    dinv_ref[...] = dinv
    bias = _graph_bias(smile_ref, wproj_ref, bproj_ref, wrep_ref)
    z = _project(x_ref[...], batch_ref, wh_ref, bias)
    zt_ref[...] = z * dinv


def _tc_layer0(degp, x, batch2, smile2, wproj, bproj2, wrep, wh):
    return pl.pallas_call(
        _layer0_body,
        grid=(NRB,),
        in_specs=[
            pl.BlockSpec((1, NW, RB), lambda j: (j, 0, 0)),
            pl.BlockSpec((RB, D), lambda j: (j, 0)),
            pl.BlockSpec((RB, 1), lambda j: (j, 0)),
            pl.BlockSpec((G, 768), lambda j: (0, 0)),
            pl.BlockSpec((768, 16), lambda j: (0, 0)),
            pl.BlockSpec((1, 16), lambda j: (0, 0)),
            pl.BlockSpec((16, D), lambda j: (0, 0)),
            pl.BlockSpec((D, D), lambda j: (0, 0)),
        ],
        out_specs=[
            pl.BlockSpec((RB, 1), lambda j: (j, 0)),
            pl.BlockSpec((RB, D), lambda j: (j, 0)),
        ],
        out_shape=[
            jax.ShapeDtypeStruct((N, 1), jnp.float32),
            jax.ShapeDtypeStruct((N, D), jnp.float32),
        ],
    )(degp, x, batch2, smile2, wproj, bproj2, wrep, wh)


def _post(acc2_ref, zt_ref, dinv_ref, bconv_ref, gsc_ref, bsh_ref):
    a = acc2_ref[...]
    pre = (a[0] + a[1] - zt_ref[...]) * dinv_ref[...] + bconv_ref[...]
    return jnp.maximum(pre, 0.0) * gsc_ref[...] + bsh_ref[...]


def _mid_body(acc2_ref, zt_ref, dinv_ref, batch_ref, smile_ref, wproj_ref,
              bproj_ref, wrep_ref, wh_ref, bconv_ref, gsc_ref, bsh_ref,
              ztn_ref):
    h = _post(acc2_ref, zt_ref, dinv_ref, bconv_ref, gsc_ref, bsh_ref)
    bias = _graph_bias(smile_ref, wproj_ref, bproj_ref, wrep_ref)
    z = _project(h, batch_ref, wh_ref, bias)
    ztn_ref[...] = z * dinv_ref[...]


def _tc_mid(acc2, zt, dinv, batch2, smile2, wproj, bproj2, wrep, wh, bconv,
            gsc, bsh):
    return pl.pallas_call(
        _mid_body,
        grid=(NRB,),
        in_specs=[
            pl.BlockSpec((NC, RB, D), lambda j: (0, j, 0)),
            pl.BlockSpec((RB, D), lambda j: (j, 0)),
            pl.BlockSpec((RB, 1), lambda j: (j, 0)),
            pl.BlockSpec((RB, 1), lambda j: (j, 0)),
            pl.BlockSpec((G, 768), lambda j: (0, 0)),
            pl.BlockSpec((768, 16), lambda j: (0, 0)),
            pl.BlockSpec((1, 16), lambda j: (0, 0)),
            pl.BlockSpec((16, D), lambda j: (0, 0)),
            pl.BlockSpec((D, D), lambda j: (0, 0)),
            pl.BlockSpec((1, D), lambda j: (0, 0)),
            pl.BlockSpec((1, D), lambda j: (0, 0)),
            pl.BlockSpec((1, D), lambda j: (0, 0)),
        ],
        out_specs=pl.BlockSpec((RB, D), lambda j: (j, 0)),
        out_shape=jax.ShapeDtypeStruct((N, D), jnp.float32),
    )(acc2, zt, dinv, batch2, smile2, wproj, bproj2, wrep, wh, bconv, gsc,
      bsh)


def _final_body(acc2_ref, zt_ref, dinv_ref, batch_ref, bconv_ref, gsc_ref,
                bsh_ref, lin1_ref, b1_ref, lin2_ref, b2_ref, out_ref,
                sums_ref, cnts_ref):
    j = pl.program_id(0)
    h = _post(acc2_ref, zt_ref, dinv_ref, bconv_ref, gsc_ref, bsh_ref)
    oh = (batch_ref[...] ==
          lax.broadcasted_iota(jnp.int32, (RB, G), 1)).astype(jnp.float32)
    contrib = lax.dot_general(oh, h, (((0,), (0,)), ((), ())),
                              preferred_element_type=jnp.float32)
    ccnt = lax.dot_general(oh, jnp.ones((RB, D), jnp.float32),
                           (((0,), (0,)), ((), ())),
                           preferred_element_type=jnp.float32)

    @pl.when(j == 0)
    def _():
        sums_ref[...] = contrib
        cnts_ref[...] = ccnt

    @pl.when(j > 0)
    def _():
        sums_ref[...] = sums_ref[...] + contrib
        cnts_ref[...] = cnts_ref[...] + ccnt

    @pl.when(j == NRB - 1)
    def _():
        pooled = sums_ref[...] / jnp.maximum(cnts_ref[...], 1.0)
        o = jnp.maximum(
            lax.dot_general(pooled, lin1_ref[...], (((1,), (0,)), ((), ())),
                            preferred_element_type=jnp.float32)
            + b1_ref[...], 0.0)
        out_ref[...] = lax.dot_general(o, lin2_ref[...],
                                       (((1,), (0,)), ((), ())),
                                       preferred_element_type=jnp.float32) \
            + b2_ref[...]


def _tc_final(acc2, zt, dinv, batch2, bconv, gsc, bsh, lin1, b1, lin2, b2):
    return pl.pallas_call(
        _final_body,
        grid=(NRB,),
        in_specs=[
            pl.BlockSpec((NC, RB, D), lambda j: (0, j, 0)),
            pl.BlockSpec((RB, D), lambda j: (j, 0)),
            pl.BlockSpec((RB, 1), lambda j: (j, 0)),
            pl.BlockSpec((RB, 1), lambda j: (j, 0)),
            pl.BlockSpec((1, D), lambda j: (0, 0)),
            pl.BlockSpec((1, D), lambda j: (0, 0)),
            pl.BlockSpec((1, D), lambda j: (0, 0)),
            pl.BlockSpec((D, D), lambda j: (0, 0)),
            pl.BlockSpec((1, D), lambda j: (0, 0)),
            pl.BlockSpec((D, 16), lambda j: (0, 0)),
            pl.BlockSpec((1, 16), lambda j: (0, 0)),
        ],
        out_specs=pl.BlockSpec((G, 16), lambda j: (0, 0)),
        out_shape=jax.ShapeDtypeStruct((G, 16), jnp.float32),
        scratch_shapes=[
            pltpu.VMEM((G, D), jnp.float32),
            pltpu.VMEM((G, D), jnp.float32),
        ],
    )(acc2, zt, dinv, batch2, bconv, gsc, bsh, lin1, b1, lin2, b2)


def kernel(x, edge_index, batch_indice, smile_llm, params):
    src = edge_index[0].astype(jnp.int32)
    dst = edge_index[1].astype(jnp.int32)
    dst_flat = dst.reshape(NW, EP)
    src3 = src.reshape(NW, NCHUNK, CH)
    dst3 = dst.reshape(NW, NCHUNK, CH)
    batch2 = batch_indice.astype(jnp.int32).reshape(N, 1)
    smile2 = smile_llm.reshape(G, 768)

    pad10 = lambda a: jnp.pad(a, [(0, 0)] * (a.ndim - 1) + [(0, 6)])
    wproj = pad10(params['W_proj'])
    bproj2 = pad10(params['b_proj'].reshape(1, 10))
    lin2 = pad10(params['lin2_W'])
    b2 = pad10(params['lin2_b'].reshape(1, 10))
    b1 = params['lin1_b'].reshape(1, D)
    bn_scale = 1.0 / jnp.sqrt(jnp.float32(1.0 + EPS))

    degp = _deg_kernel(dst_flat)

    c0 = params['convs'][0]
    wrep0 = jnp.pad(c0['W'][D:], [(0, 6), (0, 0)])
    dinv, zt = _tc_layer0(degp, x, batch2, smile2, wproj, bproj2, wrep0,
                          c0['W'][:D])

    for i in range(1, 6):
        acc2 = _mp_kernel(zt, src3, dst3)
        ci = params['convs'][i]
        bni = params['bns'][i - 1]
        cprev = params['convs'][i - 1]
        wrep = jnp.pad(ci['W'][D:], [(0, 6), (0, 0)])
        zt = _tc_mid(acc2, zt, dinv, batch2, smile2, wproj, bproj2, wrep,
                     ci['W'][:D], cprev['b'].reshape(1, D),
                     (bni['g'] * bn_scale).reshape(1, D),
                     bni['b'].reshape(1, D))

    acc2 = _mp_kernel(zt, src3, dst3)
    c5 = params['convs'][5]
    bn5 = params['bns'][5]
    out16 = _tc_final(acc2, zt, dinv, batch2, c5['b'].reshape(1, D),
                      (bn5['g'] * bn_scale).reshape(1, D),
                      bn5['b'].reshape(1, D), params['lin1_W'], b1, lin2, b2)
    return out16[:, :10]


# speedup vs baseline: 17.3515x; 1.1580x over previous
import functools

import jax
import jax.numpy as jnp
from jax import lax
from jax.experimental import pallas as pl
from jax.experimental.pallas import tpu as pltpu
from jax.experimental.pallas import tpu_sc as plsc

N = 10000
E = 320000
G = 64
D = 128
NC = 2
NS = 16
NW = NC * NS
EP = E // NW
CH = 100
NCHUNK = EP // CH
RB = 1000
NRB = N // RB
RPS = N // NS
EPS = 1e-5

_sc_mesh = plsc.VectorSubcoreMesh(core_axis_name="c", subcore_axis_name="s")


def _deg_body(dst_hbm, degp_hbm, dstv, degv):
    c = lax.axis_index("c")
    s = lax.axis_index("s")
    wid = c * NS + s
    pltpu.sync_copy(dst_hbm.at[wid], dstv)

    def zero(i, carry):
        degv[pl.ds(i * 16, 16)] = jnp.zeros((16,), jnp.float32)
        return carry

    lax.fori_loop(0, N // 16, zero, 0)
    ones = jnp.ones((16,), jnp.float32)

    def acc(i, carry):
        idx = dstv[pl.ds(i * 16, 16)]
        plsc.addupdate_scatter(degv, [idx], ones)
        return carry

    lax.fori_loop(0, EP // 16, acc, 0)
    for jb in range(NRB):
        pltpu.sync_copy(degv.at[pl.ds(jb * RB, RB)], degp_hbm.at[jb, wid])


_deg_kernel = functools.partial(
    pl.kernel,
    out_type=jax.ShapeDtypeStruct((NRB, NW, RB), jnp.float32),
    mesh=_sc_mesh,
    compiler_params=pltpu.CompilerParams(needs_layout_passes=False, use_tc_tiling_on_sc=False),
    scratch_types=[
        pltpu.VMEM((EP,), jnp.int32),
        pltpu.VMEM((N,), jnp.float32),
    ],
)(_deg_body)


NB = 2
NOUT = NCHUNK // NB


def _mp_body(zt_hbm, src_hbm, dst_hbm, acc2_hbm, srcv, dstv, r0, r1,
             acc_sh, g0, g1, s0, s1):
    c = lax.axis_index("c")
    s = lax.axis_index("s")
    wid = c * NS + s
    rows = [r0, r1]
    gsem = [g0, g1]
    ssem = [s0, s1]
    pltpu.sync_copy(src_hbm.at[wid], srcv)
    pltpu.sync_copy(dst_hbm.at[wid], dstv)
    pltpu.sync_copy(zt_hbm.at[pl.ds(s * RPS, RPS)],
                    acc_sh.at[pl.ds(s * RPS, RPS)])
    plsc.subcore_barrier()

    def gather(j, b):
        return pltpu.async_copy(zt_hbm.at[srcv.at[j]], rows[b], gsem[b])

    def scatter(j, b):
        return pltpu.async_copy(rows[b], acc_sh.at[dstv.at[j]], ssem[b],
                                add=True)

    for b in range(NB):
        gather(b, b)

    def outer(i, carry):
        for b in range(NB):
            j = i * NB + b
            pltpu.make_async_copy(zt_hbm.at[srcv.at[j]], rows[b],
                                  gsem[b]).wait()
            scatter(j, b)
        for b in range(NB):
            j = i * NB + b
            pltpu.make_async_copy(rows[b], acc_sh.at[dstv.at[j]],
                                  ssem[b]).wait()

            @pl.when(i < NOUT - 1)
            def _():
                gather(j + NB, b)

        return carry

    lax.fori_loop(0, NOUT, outer, 0)
    plsc.subcore_barrier()
    pltpu.sync_copy(acc_sh.at[pl.ds(s * RPS, RPS)],
                    acc2_hbm.at[c, pl.ds(s * RPS, RPS)])


_mp_kernel = functools.partial(
    pl.kernel,
    out_type=jax.ShapeDtypeStruct((NC, N, D), jnp.float32),
    mesh=_sc_mesh,
    compiler_params=pltpu.CompilerParams(needs_layout_passes=False, use_tc_tiling_on_sc=False),
    scratch_types=[
        pltpu.VMEM((NCHUNK, CH), jnp.int32),
        pltpu.VMEM((NCHUNK, CH), jnp.int32),
        pltpu.VMEM((CH, D), jnp.float32),
        pltpu.VMEM((CH, D), jnp.float32),
        pltpu.VMEM_SHARED((N, D), jnp.float32),
        pltpu.SemaphoreType.DMA,
        pltpu.SemaphoreType.DMA,
        pltpu.SemaphoreType.DMA,
        pltpu.SemaphoreType.DMA,
    ],
)(_mp_body)


def _graph_bias(smile_ref, wproj_ref, bproj_ref, wrep_ref):
    s = jnp.maximum(
        lax.dot_general(smile_ref[...], wproj_ref[...],
                        (((1,), (0,)), ((), ())),
                        preferred_element_type=jnp.float32)
        + bproj_ref[...], 0.0)
    return lax.dot_general(s, wrep_ref[...], (((1,), (0,)), ((), ())),
                           preferred_element_type=jnp.float32)


def _project(h, batch_ref, wh_ref, bias):
    oh = (batch_ref[...] ==
          lax.broadcasted_iota(jnp.int32, (h.shape[0], G), 1)
          ).astype(jnp.float32)
    z = lax.dot_general(h, wh_ref[...], (((1,), (0,)), ((), ())),
                        preferred_element_type=jnp.float32)
    z = z + lax.dot_general(oh, bias, (((1,), (0,)), ((), ())),
                            preferred_element_type=jnp.float32)
    return z


def _layer0_body(degp_ref, x_ref, batch_ref, smile_ref, wproj_ref, bproj_ref,
                 wrep_ref, wh_ref, dinv_ref, zt_ref):
    degsum = lax.dot_general(degp_ref[...].reshape(NW, RB),
                             jnp.ones((NW, 1), jnp.float32),
                             (((0,), (0,)), ((), ())),
                             preferred_element_type=jnp.float32)
    dinv = lax.rsqrt(degsum + 1.0)
    dinv_ref[...] = dinv
    bias = _graph_bias(smile_ref, wproj_ref, bproj_ref, wrep_ref)
    z = _project(x_ref[...], batch_ref, wh_ref, bias)
    zt_ref[...] = z * dinv


def _tc_layer0(degp, x, batch2, smile2, wproj, bproj2, wrep, wh):
    return pl.pallas_call(
        _layer0_body,
        grid=(NRB,),
        in_specs=[
            pl.BlockSpec((1, NW, RB), lambda j: (j, 0, 0)),
            pl.BlockSpec((RB, D), lambda j: (j, 0)),
            pl.BlockSpec((RB, 1), lambda j: (j, 0)),
            pl.BlockSpec((G, 768), lambda j: (0, 0)),
            pl.BlockSpec((768, 16), lambda j: (0, 0)),
            pl.BlockSpec((1, 16), lambda j: (0, 0)),
            pl.BlockSpec((16, D), lambda j: (0, 0)),
            pl.BlockSpec((D, D), lambda j: (0, 0)),
        ],
        out_specs=[
            pl.BlockSpec((RB, 1), lambda j: (j, 0)),
            pl.BlockSpec((RB, D), lambda j: (j, 0)),
        ],
        out_shape=[
            jax.ShapeDtypeStruct((N, 1), jnp.float32),
            jax.ShapeDtypeStruct((N, D), jnp.float32),
        ],
    )(degp, x, batch2, smile2, wproj, bproj2, wrep, wh)


def _post(acc2_ref, zt_ref, dinv_ref, bconv_ref, gsc_ref, bsh_ref):
    a = acc2_ref[...]
    pre = (a[0] + a[1] - zt_ref[...]) * dinv_ref[...] + bconv_ref[...]
    return jnp.maximum(pre, 0.0) * gsc_ref[...] + bsh_ref[...]


def _mid_body(acc2_ref, zt_ref, dinv_ref, batch_ref, smile_ref, wproj_ref,
              bproj_ref, wrep_ref, wh_ref, bconv_ref, gsc_ref, bsh_ref,
              ztn_ref):
    h = _post(acc2_ref, zt_ref, dinv_ref, bconv_ref, gsc_ref, bsh_ref)
    bias = _graph_bias(smile_ref, wproj_ref, bproj_ref, wrep_ref)
    z = _project(h, batch_ref, wh_ref, bias)
    ztn_ref[...] = z * dinv_ref[...]


def _tc_mid(acc2, zt, dinv, batch2, smile2, wproj, bproj2, wrep, wh, bconv,
            gsc, bsh):
    return pl.pallas_call(
        _mid_body,
        grid=(NRB,),
        in_specs=[
            pl.BlockSpec((NC, RB, D), lambda j: (0, j, 0)),
            pl.BlockSpec((RB, D), lambda j: (j, 0)),
            pl.BlockSpec((RB, 1), lambda j: (j, 0)),
            pl.BlockSpec((RB, 1), lambda j: (j, 0)),
            pl.BlockSpec((G, 768), lambda j: (0, 0)),
            pl.BlockSpec((768, 16), lambda j: (0, 0)),
            pl.BlockSpec((1, 16), lambda j: (0, 0)),
            pl.BlockSpec((16, D), lambda j: (0, 0)),
            pl.BlockSpec((D, D), lambda j: (0, 0)),
            pl.BlockSpec((1, D), lambda j: (0, 0)),
            pl.BlockSpec((1, D), lambda j: (0, 0)),
            pl.BlockSpec((1, D), lambda j: (0, 0)),
        ],
        out_specs=pl.BlockSpec((RB, D), lambda j: (j, 0)),
        out_shape=jax.ShapeDtypeStruct((N, D), jnp.float32),
    )(acc2, zt, dinv, batch2, smile2, wproj, bproj2, wrep, wh, bconv, gsc,
      bsh)


def _final_body(acc2_ref, zt_ref, dinv_ref, batch_ref, bconv_ref, gsc_ref,
                bsh_ref, lin1_ref, b1_ref, lin2_ref, b2_ref, out_ref,
                sums_ref, cnts_ref):
    j = pl.program_id(0)
    h = _post(acc2_ref, zt_ref, dinv_ref, bconv_ref, gsc_ref, bsh_ref)
    oh = (batch_ref[...] ==
          lax.broadcasted_iota(jnp.int32, (RB, G), 1)).astype(jnp.float32)
    contrib = lax.dot_general(oh, h, (((0,), (0,)), ((), ())),
                              preferred_element_type=jnp.float32)
    ccnt = lax.dot_general(oh, jnp.ones((RB, D), jnp.float32),
                           (((0,), (0,)), ((), ())),
                           preferred_element_type=jnp.float32)

    @pl.when(j == 0)
    def _():
        sums_ref[...] = contrib
        cnts_ref[...] = ccnt

    @pl.when(j > 0)
    def _():
        sums_ref[...] = sums_ref[...] + contrib
        cnts_ref[...] = cnts_ref[...] + ccnt

    @pl.when(j == NRB - 1)
    def _():
        pooled = sums_ref[...] / jnp.maximum(cnts_ref[...], 1.0)
        o = jnp.maximum(
            lax.dot_general(pooled, lin1_ref[...], (((1,), (0,)), ((), ())),
                            preferred_element_type=jnp.float32)
            + b1_ref[...], 0.0)
        out_ref[...] = lax.dot_general(o, lin2_ref[...],
                                       (((1,), (0,)), ((), ())),
                                       preferred_element_type=jnp.float32) \
            + b2_ref[...]


def _tc_final(acc2, zt, dinv, batch2, bconv, gsc, bsh, lin1, b1, lin2, b2):
    return pl.pallas_call(
        _final_body,
        grid=(NRB,),
        in_specs=[
            pl.BlockSpec((NC, RB, D), lambda j: (0, j, 0)),
            pl.BlockSpec((RB, D), lambda j: (j, 0)),
            pl.BlockSpec((RB, 1), lambda j: (j, 0)),
            pl.BlockSpec((RB, 1), lambda j: (j, 0)),
            pl.BlockSpec((1, D), lambda j: (0, 0)),
            pl.BlockSpec((1, D), lambda j: (0, 0)),
            pl.BlockSpec((1, D), lambda j: (0, 0)),
            pl.BlockSpec((D, D), lambda j: (0, 0)),
            pl.BlockSpec((1, D), lambda j: (0, 0)),
            pl.BlockSpec((D, 16), lambda j: (0, 0)),
            pl.BlockSpec((1, 16), lambda j: (0, 0)),
        ],
        out_specs=pl.BlockSpec((G, 16), lambda j: (0, 0)),
        out_shape=jax.ShapeDtypeStruct((G, 16), jnp.float32),
        scratch_shapes=[
            pltpu.VMEM((G, D), jnp.float32),
            pltpu.VMEM((G, D), jnp.float32),
        ],
    )(acc2, zt, dinv, batch2, bconv, gsc, bsh, lin1, b1, lin2, b2)


def kernel(x, edge_index, batch_indice, smile_llm, params):
    src = edge_index[0].astype(jnp.int32)
    dst = edge_index[1].astype(jnp.int32)
    dst_flat = dst.reshape(NW, EP)
    src3 = src.reshape(NW, NCHUNK, CH)
    dst3 = dst.reshape(NW, NCHUNK, CH)
    batch2 = batch_indice.astype(jnp.int32).reshape(N, 1)
    smile2 = smile_llm.reshape(G, 768)

    pad10 = lambda a: jnp.pad(a, [(0, 0)] * (a.ndim - 1) + [(0, 6)])
    wproj = pad10(params['W_proj'])
    bproj2 = pad10(params['b_proj'].reshape(1, 10))
    lin2 = pad10(params['lin2_W'])
    b2 = pad10(params['lin2_b'].reshape(1, 10))
    b1 = params['lin1_b'].reshape(1, D)
    bn_scale = 1.0 / jnp.sqrt(jnp.float32(1.0 + EPS))

    degp = _deg_kernel(dst_flat)

    c0 = params['convs'][0]
    wrep0 = jnp.pad(c0['W'][D:], [(0, 6), (0, 0)])
    dinv, zt = _tc_layer0(degp, x, batch2, smile2, wproj, bproj2, wrep0,
                          c0['W'][:D])

    for i in range(1, 6):
        acc2 = _mp_kernel(zt, src3, dst3)
        ci = params['convs'][i]
        bni = params['bns'][i - 1]
        cprev = params['convs'][i - 1]
        wrep = jnp.pad(ci['W'][D:], [(0, 6), (0, 0)])
        zt = _tc_mid(acc2, zt, dinv, batch2, smile2, wproj, bproj2, wrep,
                     ci['W'][:D], cprev['b'].reshape(1, D),
                     (bni['g'] * bn_scale).reshape(1, D),
                     bni['b'].reshape(1, D))

    acc2 = _mp_kernel(zt, src3, dst3)
    c5 = params['convs'][5]
    bn5 = params['bns'][5]
    out16 = _tc_final(acc2, zt, dinv, batch2, c5['b'].reshape(1, D),
                      (bn5['g'] * bn_scale).reshape(1, D),
                      bn5['b'].reshape(1, D), params['lin1_W'], b1, lin2, b2)
    return out16[:, :10]


# speedup vs baseline: 21.2980x; 1.2274x over previous
import functools

import jax
import jax.numpy as jnp
from jax import lax
from jax.experimental import pallas as pl
from jax.experimental.pallas import tpu as pltpu
from jax.experimental.pallas import tpu_sc as plsc

N = 10000
E = 320000
G = 64
D = 128
NC = 2
NS = 16
NW = NC * NS
EP = E // NW
CH = 50
NCHUNK = EP // CH
RB = 1000
NRB = N // RB
RPS = N // NS
EPS = 1e-5

_sc_mesh = plsc.VectorSubcoreMesh(core_axis_name="c", subcore_axis_name="s")


def _deg_body(dst_hbm, degp_hbm, dstv, degv):
    c = lax.axis_index("c")
    s = lax.axis_index("s")
    wid = c * NS + s
    pltpu.sync_copy(dst_hbm.at[wid], dstv)

    def zero(i, carry):
        degv[pl.ds(i * 16, 16)] = jnp.zeros((16,), jnp.float32)
        return carry

    lax.fori_loop(0, N // 16, zero, 0)
    ones = jnp.ones((16,), jnp.float32)

    def acc(i, carry):
        idx = dstv[pl.ds(i * 16, 16)]
        plsc.addupdate_scatter(degv, [idx], ones)
        return carry

    lax.fori_loop(0, EP // 16, acc, 0)
    for jb in range(NRB):
        pltpu.sync_copy(degv.at[pl.ds(jb * RB, RB)], degp_hbm.at[jb, wid])


_deg_kernel = functools.partial(
    pl.kernel,
    out_type=jax.ShapeDtypeStruct((NRB, NW, RB), jnp.float32),
    mesh=_sc_mesh,
    compiler_params=pltpu.CompilerParams(needs_layout_passes=False, use_tc_tiling_on_sc=False),
    scratch_types=[
        pltpu.VMEM((EP,), jnp.int32),
        pltpu.VMEM((N,), jnp.float32),
    ],
)(_deg_body)


NB = 4
NOUT = NCHUNK // NB


def _mp_body(zt_hbm, src_hbm, dst_hbm, acc2_hbm, srcv, dstv, r0, r1, r2, r3,
             acc_sh, g0, g1, g2, g3, s0, s1, s2, s3):
    c = lax.axis_index("c")
    s = lax.axis_index("s")
    wid = c * NS + s
    rows = [r0, r1, r2, r3]
    gsem = [g0, g1, g2, g3]
    ssem = [s0, s1, s2, s3]
    pltpu.async_copy(src_hbm.at[wid], srcv, g0)
    pltpu.async_copy(dst_hbm.at[wid], dstv, g1)
    pltpu.async_copy(zt_hbm.at[pl.ds(s * RPS, RPS)],
                     acc_sh.at[pl.ds(s * RPS, RPS)], g2)
    pltpu.make_async_copy(src_hbm.at[wid], srcv, g0).wait()
    pltpu.make_async_copy(dst_hbm.at[wid], dstv, g1).wait()
    pltpu.make_async_copy(zt_hbm.at[pl.ds(s * RPS, RPS)],
                          acc_sh.at[pl.ds(s * RPS, RPS)], g2).wait()
    plsc.subcore_barrier()

    def gather(j, b):
        return pltpu.async_copy(zt_hbm.at[srcv.at[j]], rows[b], gsem[b])

    def scatter(j, b):
        return pltpu.async_copy(rows[b], acc_sh.at[dstv.at[j]], ssem[b],
                                add=True)

    for b in range(NB):
        gather(b, b)

    def outer(i, carry):
        for b in range(NB):
            j = i * NB + b
            pltpu.make_async_copy(zt_hbm.at[srcv.at[j]], rows[b],
                                  gsem[b]).wait()
            scatter(j, b)
        for b in range(NB):
            j = i * NB + b
            pltpu.make_async_copy(rows[b], acc_sh.at[dstv.at[j]],
                                  ssem[b]).wait()

            @pl.when(i < NOUT - 1)
            def _():
                gather(j + NB, b)

        return carry

    lax.fori_loop(0, NOUT, outer, 0)
    plsc.subcore_barrier()
    pltpu.sync_copy(acc_sh.at[pl.ds(s * RPS, RPS)],
                    acc2_hbm.at[c, pl.ds(s * RPS, RPS)])


_mp_kernel = functools.partial(
    pl.kernel,
    out_type=jax.ShapeDtypeStruct((NC, N, D), jnp.float32),
    mesh=_sc_mesh,
    compiler_params=pltpu.CompilerParams(needs_layout_passes=False, use_tc_tiling_on_sc=False),
    scratch_types=[
        pltpu.VMEM((NCHUNK, CH), jnp.int32),
        pltpu.VMEM((NCHUNK, CH), jnp.int32),
        pltpu.VMEM((CH, D), jnp.float32),
        pltpu.VMEM((CH, D), jnp.float32),
        pltpu.VMEM((CH, D), jnp.float32),
        pltpu.VMEM((CH, D), jnp.float32),
        pltpu.VMEM_SHARED((N, D), jnp.float32),
        pltpu.SemaphoreType.DMA,
        pltpu.SemaphoreType.DMA,
        pltpu.SemaphoreType.DMA,
        pltpu.SemaphoreType.DMA,
        pltpu.SemaphoreType.DMA,
        pltpu.SemaphoreType.DMA,
        pltpu.SemaphoreType.DMA,
        pltpu.SemaphoreType.DMA,
    ],
)(_mp_body)


def _graph_bias(smile_ref, wproj_ref, bproj_ref, wrep_ref):
    s = jnp.maximum(
        lax.dot_general(smile_ref[...], wproj_ref[...],
                        (((1,), (0,)), ((), ())),
                        preferred_element_type=jnp.float32)
        + bproj_ref[...], 0.0)
    return lax.dot_general(s, wrep_ref[...], (((1,), (0,)), ((), ())),
                           preferred_element_type=jnp.float32)


def _project(h, batch_ref, wh_ref, bias):
    oh = (batch_ref[...] ==
          lax.broadcasted_iota(jnp.int32, (h.shape[0], G), 1)
          ).astype(jnp.float32)
    z = lax.dot_general(h, wh_ref[...], (((1,), (0,)), ((), ())),
                        preferred_element_type=jnp.float32)
    z = z + lax.dot_general(oh, bias, (((1,), (0,)), ((), ())),
                            preferred_element_type=jnp.float32)
    return z


def _layer0_body(degp_ref, x_ref, batch_ref, smile_ref, wproj_ref, bproj_ref,
                 wrep_ref, wh_ref, dinv_ref, zt_ref):
    degsum = lax.dot_general(degp_ref[...].reshape(NW, RB),
                             jnp.ones((NW, 1), jnp.float32),
                             (((0,), (0,)), ((), ())),
                             preferred_element_type=jnp.float32)
    dinv = lax.rsqrt(degsum + 1.0)
    dinv_ref[...] = dinv
    bias = _graph_bias(smile_ref, wproj_ref, bproj_ref, wrep_ref)
    z = _project(x_ref[...], batch_ref, wh_ref, bias)
    zt_ref[...] = z * dinv


def _tc_layer0(degp, x, batch2, smile2, wproj, bproj2, wrep, wh):
    return pl.pallas_call(
        _layer0_body,
        grid=(NRB,),
        in_specs=[
            pl.BlockSpec((1, NW, RB), lambda j: (j, 0, 0)),
            pl.BlockSpec((RB, D), lambda j: (j, 0)),
            pl.BlockSpec((RB, 1), lambda j: (j, 0)),
            pl.BlockSpec((G, 768), lambda j: (0, 0)),
            pl.BlockSpec((768, 16), lambda j: (0, 0)),
            pl.BlockSpec((1, 16), lambda j: (0, 0)),
            pl.BlockSpec((16, D), lambda j: (0, 0)),
            pl.BlockSpec((D, D), lambda j: (0, 0)),
        ],
        out_specs=[
            pl.BlockSpec((RB, 1), lambda j: (j, 0)),
            pl.BlockSpec((RB, D), lambda j: (j, 0)),
        ],
        out_shape=[
            jax.ShapeDtypeStruct((N, 1), jnp.float32),
            jax.ShapeDtypeStruct((N, D), jnp.float32),
        ],
    )(degp, x, batch2, smile2, wproj, bproj2, wrep, wh)


def _post(acc2_ref, zt_ref, dinv_ref, bconv_ref, gsc_ref, bsh_ref):
    a = acc2_ref[...]
    pre = (a[0] + a[1] - zt_ref[...]) * dinv_ref[...] + bconv_ref[...]
    return jnp.maximum(pre, 0.0) * gsc_ref[...] + bsh_ref[...]


def _mid_body(acc2_ref, zt_ref, dinv_ref, batch_ref, smile_ref, wproj_ref,
              bproj_ref, wrep_ref, wh_ref, bconv_ref, gsc_ref, bsh_ref,
              ztn_ref):
    h = _post(acc2_ref, zt_ref, dinv_ref, bconv_ref, gsc_ref, bsh_ref)
    bias = _graph_bias(smile_ref, wproj_ref, bproj_ref, wrep_ref)
    z = _project(h, batch_ref, wh_ref, bias)
    ztn_ref[...] = z * dinv_ref[...]


def _tc_mid(acc2, zt, dinv, batch2, smile2, wproj, bproj2, wrep, wh, bconv,
            gsc, bsh):
    return pl.pallas_call(
        _mid_body,
        grid=(NRB,),
        in_specs=[
            pl.BlockSpec((NC, RB, D), lambda j: (0, j, 0)),
            pl.BlockSpec((RB, D), lambda j: (j, 0)),
            pl.BlockSpec((RB, 1), lambda j: (j, 0)),
            pl.BlockSpec((RB, 1), lambda j: (j, 0)),
            pl.BlockSpec((G, 768), lambda j: (0, 0)),
            pl.BlockSpec((768, 16), lambda j: (0, 0)),
            pl.BlockSpec((1, 16), lambda j: (0, 0)),
            pl.BlockSpec((16, D), lambda j: (0, 0)),
            pl.BlockSpec((D, D), lambda j: (0, 0)),
            pl.BlockSpec((1, D), lambda j: (0, 0)),
            pl.BlockSpec((1, D), lambda j: (0, 0)),
            pl.BlockSpec((1, D), lambda j: (0, 0)),
        ],
        out_specs=pl.BlockSpec((RB, D), lambda j: (j, 0)),
        out_shape=jax.ShapeDtypeStruct((N, D), jnp.float32),
    )(acc2, zt, dinv, batch2, smile2, wproj, bproj2, wrep, wh, bconv, gsc,
      bsh)


def _final_body(acc2_ref, zt_ref, dinv_ref, batch_ref, bconv_ref, gsc_ref,
                bsh_ref, lin1_ref, b1_ref, lin2_ref, b2_ref, out_ref,
                sums_ref, cnts_ref):
    j = pl.program_id(0)
    h = _post(acc2_ref, zt_ref, dinv_ref, bconv_ref, gsc_ref, bsh_ref)
    oh = (batch_ref[...] ==
          lax.broadcasted_iota(jnp.int32, (RB, G), 1)).astype(jnp.float32)
    contrib = lax.dot_general(oh, h, (((0,), (0,)), ((), ())),
                              preferred_element_type=jnp.float32)
    ccnt = lax.dot_general(oh, jnp.ones((RB, D), jnp.float32),
                           (((0,), (0,)), ((), ())),
                           preferred_element_type=jnp.float32)

    @pl.when(j == 0)
    def _():
        sums_ref[...] = contrib
        cnts_ref[...] = ccnt

    @pl.when(j > 0)
    def _():
        sums_ref[...] = sums_ref[...] + contrib
        cnts_ref[...] = cnts_ref[...] + ccnt

    @pl.when(j == NRB - 1)
    def _():
        pooled = sums_ref[...] / jnp.maximum(cnts_ref[...], 1.0)
        o = jnp.maximum(
            lax.dot_general(pooled, lin1_ref[...], (((1,), (0,)), ((), ())),
                            preferred_element_type=jnp.float32)
            + b1_ref[...], 0.0)
        out_ref[...] = lax.dot_general(o, lin2_ref[...],
                                       (((1,), (0,)), ((), ())),
                                       preferred_element_type=jnp.float32) \
            + b2_ref[...]


def _tc_final(acc2, zt, dinv, batch2, bconv, gsc, bsh, lin1, b1, lin2, b2):
    return pl.pallas_call(
        _final_body,
        grid=(NRB,),
        in_specs=[
            pl.BlockSpec((NC, RB, D), lambda j: (0, j, 0)),
            pl.BlockSpec((RB, D), lambda j: (j, 0)),
            pl.BlockSpec((RB, 1), lambda j: (j, 0)),
            pl.BlockSpec((RB, 1), lambda j: (j, 0)),
            pl.BlockSpec((1, D), lambda j: (0, 0)),
            pl.BlockSpec((1, D), lambda j: (0, 0)),
            pl.BlockSpec((1, D), lambda j: (0, 0)),
            pl.BlockSpec((D, D), lambda j: (0, 0)),
            pl.BlockSpec((1, D), lambda j: (0, 0)),
            pl.BlockSpec((D, 16), lambda j: (0, 0)),
            pl.BlockSpec((1, 16), lambda j: (0, 0)),
        ],
        out_specs=pl.BlockSpec((G, 16), lambda j: (0, 0)),
        out_shape=jax.ShapeDtypeStruct((G, 16), jnp.float32),
        scratch_shapes=[
            pltpu.VMEM((G, D), jnp.float32),
            pltpu.VMEM((G, D), jnp.float32),
        ],
    )(acc2, zt, dinv, batch2, bconv, gsc, bsh, lin1, b1, lin2, b2)


def kernel(x, edge_index, batch_indice, smile_llm, params):
    src = edge_index[0].astype(jnp.int32)
    dst = edge_index[1].astype(jnp.int32)
    dst_flat = dst.reshape(NW, EP)
    src3 = src.reshape(NW, NCHUNK, CH)
    dst3 = dst.reshape(NW, NCHUNK, CH)
    batch2 = batch_indice.astype(jnp.int32).reshape(N, 1)
    smile2 = smile_llm.reshape(G, 768)

    pad10 = lambda a: jnp.pad(a, [(0, 0)] * (a.ndim - 1) + [(0, 6)])
    wproj = pad10(params['W_proj'])
    bproj2 = pad10(params['b_proj'].reshape(1, 10))
    lin2 = pad10(params['lin2_W'])
    b2 = pad10(params['lin2_b'].reshape(1, 10))
    b1 = params['lin1_b'].reshape(1, D)
    bn_scale = 1.0 / jnp.sqrt(jnp.float32(1.0 + EPS))

    degp = _deg_kernel(dst_flat)

    c0 = params['convs'][0]
    wrep0 = jnp.pad(c0['W'][D:], [(0, 6), (0, 0)])
    dinv, zt = _tc_layer0(degp, x, batch2, smile2, wproj, bproj2, wrep0,
                          c0['W'][:D])

    for i in range(1, 6):
        acc2 = _mp_kernel(zt, src3, dst3)
        ci = params['convs'][i]
        bni = params['bns'][i - 1]
        cprev = params['convs'][i - 1]
        wrep = jnp.pad(ci['W'][D:], [(0, 6), (0, 0)])
        zt = _tc_mid(acc2, zt, dinv, batch2, smile2, wproj, bproj2, wrep,
                     ci['W'][:D], cprev['b'].reshape(1, D),
                     (bni['g'] * bn_scale).reshape(1, D),
                     bni['b'].reshape(1, D))

    acc2 = _mp_kernel(zt, src3, dst3)
    c5 = params['convs'][5]
    bn5 = params['bns'][5]
    out16 = _tc_final(acc2, zt, dinv, batch2, c5['b'].reshape(1, D),
                      (bn5['g'] * bn_scale).reshape(1, D),
                      bn5['b'].reshape(1, D), params['lin1_W'], b1, lin2, b2)
    return out16[:, :10]


# speedup vs baseline: 22.1021x; 1.0378x over previous
import functools

import jax
import jax.numpy as jnp
from jax import lax
from jax.experimental import pallas as pl
from jax.experimental.pallas import tpu as pltpu
from jax.experimental.pallas import tpu_sc as plsc

N = 10000
E = 320000
G = 64
D = 128
LD = 768
NCL = 10
NC = 2
NS = 16
NW = NC * NS
EP = E // NW
CH = 50
NCHUNK = EP // CH
NB = 4
NOUT = NCHUNK // NB
RB = 2000
NRB = N // RB
RPS = N // NS
EPS = 1e-5

_sc_mesh = plsc.VectorSubcoreMesh(core_axis_name="c", subcore_axis_name="s")
_sc_params = pltpu.CompilerParams(needs_layout_passes=False,
                                  use_tc_tiling_on_sc=False)


def _deg_body(e4_hbm, degp_hbm, dstv, degv):
    c = lax.axis_index("c")
    s = lax.axis_index("s")
    wid = c * NS + s
    pltpu.sync_copy(e4_hbm.at[1, wid], dstv)

    def zero(i, carry):
        degv[pl.ds(i * 16, 16)] = jnp.zeros((16,), jnp.float32)
        return carry

    lax.fori_loop(0, N // 16, zero, 0)
    ones = jnp.ones((16,), jnp.float32)
    tail_mask = lax.broadcasted_iota(jnp.int32, (16,), 0) >= 14

    def acc(j, carry):
        for k in range(3):
            idx = dstv[j, pl.ds(k * 16, 16)]
            plsc.addupdate_scatter(degv, [idx], ones)
        idx = dstv[j, pl.ds(CH - 16, 16)]
        plsc.addupdate_scatter(degv, [idx], ones, mask=tail_mask)
        return carry

    lax.fori_loop(0, NCHUNK, acc, 0)
    for jb in range(NRB):
        pltpu.sync_copy(degv.at[pl.ds(jb * RB, RB)], degp_hbm.at[jb, wid])


_deg_kernel = functools.partial(
    pl.kernel,
    out_type=jax.ShapeDtypeStruct((NRB, NW, RB), jnp.float32),
    mesh=_sc_mesh,
    compiler_params=_sc_params,
    scratch_types=[
        pltpu.VMEM((NCHUNK, CH), jnp.int32),
        pltpu.VMEM((N,), jnp.float32),
    ],
)(_deg_body)


def _mp_body(zt_hbm, e4_hbm, acc2_hbm, srcv, dstv, r0, r1, r2, r3,
             acc_sh, g0, g1, g2, g3, s0, s1, s2, s3):
    c = lax.axis_index("c")
    s = lax.axis_index("s")
    wid = c * NS + s
    rows = [r0, r1, r2, r3]
    gsem = [g0, g1, g2, g3]
    ssem = [s0, s1, s2, s3]
    pltpu.async_copy(e4_hbm.at[0, wid], srcv, g0)
    pltpu.async_copy(e4_hbm.at[1, wid], dstv, g1)
    pltpu.async_copy(zt_hbm.at[pl.ds(s * RPS, RPS)],
                     acc_sh.at[pl.ds(s * RPS, RPS)], g2)
    pltpu.make_async_copy(e4_hbm.at[0, wid], srcv, g0).wait()
    pltpu.make_async_copy(e4_hbm.at[1, wid], dstv, g1).wait()
    pltpu.make_async_copy(zt_hbm.at[pl.ds(s * RPS, RPS)],
                          acc_sh.at[pl.ds(s * RPS, RPS)], g2).wait()
    plsc.subcore_barrier()

    def gather(j, b):
        return pltpu.async_copy(zt_hbm.at[srcv.at[j]], rows[b], gsem[b])

    def scatter(j, b):
        return pltpu.async_copy(rows[b], acc_sh.at[dstv.at[j]], ssem[b],
                                add=True)

    for b in range(NB):
        gather(b, b)

    def outer(i, carry):
        for b in range(NB):
            j = i * NB + b
            pltpu.make_async_copy(zt_hbm.at[srcv.at[j]], rows[b],
                                  gsem[b]).wait()
            scatter(j, b)
        for b in range(NB):
            j = i * NB + b
            pltpu.make_async_copy(rows[b], acc_sh.at[dstv.at[j]],
                                  ssem[b]).wait()

            @pl.when(i < NOUT - 1)
            def _():
                gather(j + NB, b)

        return carry

    lax.fori_loop(0, NOUT, outer, 0)
    plsc.subcore_barrier()
    pltpu.sync_copy(acc_sh.at[pl.ds(s * RPS, RPS)],
                    acc2_hbm.at[c, pl.ds(s * RPS, RPS)])


_mp_kernel = functools.partial(
    pl.kernel,
    out_type=jax.ShapeDtypeStruct((NC, N, D), jnp.float32),
    mesh=_sc_mesh,
    compiler_params=_sc_params,
    scratch_types=[
        pltpu.VMEM((NCHUNK, CH), jnp.int32),
        pltpu.VMEM((NCHUNK, CH), jnp.int32),
        pltpu.VMEM((CH, D), jnp.float32),
        pltpu.VMEM((CH, D), jnp.float32),
        pltpu.VMEM((CH, D), jnp.float32),
        pltpu.VMEM((CH, D), jnp.float32),
        pltpu.VMEM_SHARED((N, D), jnp.float32),
        pltpu.SemaphoreType.DMA,
        pltpu.SemaphoreType.DMA,
        pltpu.SemaphoreType.DMA,
        pltpu.SemaphoreType.DMA,
        pltpu.SemaphoreType.DMA,
        pltpu.SemaphoreType.DMA,
        pltpu.SemaphoreType.DMA,
        pltpu.SemaphoreType.DMA,
    ],
)(_mp_body)


def _graph_bias(smile_ref, wproj_ref, bproj_ref, wrep_ref):
    s = jnp.maximum(
        lax.dot_general(smile_ref[...].reshape(G, LD), wproj_ref[...],
                        (((1,), (0,)), ((), ())),
                        preferred_element_type=jnp.float32)
        + bproj_ref[...], 0.0)
    return lax.dot_general(s, wrep_ref[...], (((1,), (0,)), ((), ())),
                           preferred_element_type=jnp.float32)


def _project(h, batch_ref, wh_ref, bias):
    oh = (batch_ref[...] ==
          lax.broadcasted_iota(jnp.int32, (h.shape[0], G), 1)
          ).astype(jnp.float32)
    z = lax.dot_general(h, wh_ref[...], (((1,), (0,)), ((), ())),
                        preferred_element_type=jnp.float32)
    z = z + lax.dot_general(oh, bias, (((1,), (0,)), ((), ())),
                            preferred_element_type=jnp.float32)
    return z


def _layer0_body(degp_ref, x_ref, batch_ref, smile_ref, wproj_ref, bproj_ref,
                 wrep_ref, wh_ref, dinv_ref, zt_ref):
    degsum = lax.dot_general(degp_ref[...].reshape(NW, RB),
                             jnp.ones((NW, 1), jnp.float32),
                             (((0,), (0,)), ((), ())),
                             preferred_element_type=jnp.float32)
    dinv = lax.rsqrt(degsum + 1.0)
    dinv_ref[...] = dinv
    bias = _graph_bias(smile_ref, wproj_ref, bproj_ref, wrep_ref)
    z = _project(x_ref[...], batch_ref, wh_ref, bias)
    zt_ref[...] = z * dinv


def _tc_layer0(degp, x, batch2, smile3, wproj, bproj2, wrep, wh):
    return pl.pallas_call(
        _layer0_body,
        grid=(NRB,),
        in_specs=[
            pl.BlockSpec((1, NW, RB), lambda j: (j, 0, 0)),
            pl.BlockSpec((RB, D), lambda j: (j, 0)),
            pl.BlockSpec((RB, 1), lambda j: (j, 0)),
            pl.BlockSpec((G, 1, LD), lambda j: (0, 0, 0)),
            pl.BlockSpec((LD, NCL), lambda j: (0, 0)),
            pl.BlockSpec((1, NCL), lambda j: (0, 0)),
            pl.BlockSpec((NCL, D), lambda j: (0, 0)),
            pl.BlockSpec((D, D), lambda j: (0, 0)),
        ],
        out_specs=[
            pl.BlockSpec((RB, 1), lambda j: (j, 0)),
            pl.BlockSpec((RB, D), lambda j: (j, 0)),
        ],
        out_shape=[
            jax.ShapeDtypeStruct((N, 1), jnp.float32),
            jax.ShapeDtypeStruct((N, D), jnp.float32),
        ],
    )(degp, x, batch2, smile3, wproj, bproj2, wrep, wh)


def _post(acc2_ref, zt_ref, dinv_ref, bconv_ref, gsc_ref, bsh_ref):
    a = acc2_ref[...]
    pre = (a[0] + a[1] - zt_ref[...]) * dinv_ref[...] + bconv_ref[...]
    return jnp.maximum(pre, 0.0) * gsc_ref[...] + bsh_ref[...]


def _mid_body(acc2_ref, zt_ref, dinv_ref, batch_ref, smile_ref, wproj_ref,
              bproj_ref, wrep_ref, wh_ref, bconv_ref, gsc_ref, bsh_ref,
              ztn_ref):
    h = _post(acc2_ref, zt_ref, dinv_ref, bconv_ref, gsc_ref, bsh_ref)
    bias = _graph_bias(smile_ref, wproj_ref, bproj_ref, wrep_ref)
    z = _project(h, batch_ref, wh_ref, bias)
    ztn_ref[...] = z * dinv_ref[...]


def _tc_mid(acc2, zt, dinv, batch2, smile3, wproj, bproj2, wrep, wh, bconv,
            gsc, bsh):
    return pl.pallas_call(
        _mid_body,
        grid=(NRB,),
        in_specs=[
            pl.BlockSpec((NC, RB, D), lambda j: (0, j, 0)),
            pl.BlockSpec((RB, D), lambda j: (j, 0)),
            pl.BlockSpec((RB, 1), lambda j: (j, 0)),
            pl.BlockSpec((RB, 1), lambda j: (j, 0)),
            pl.BlockSpec((G, 1, LD), lambda j: (0, 0, 0)),
            pl.BlockSpec((LD, NCL), lambda j: (0, 0)),
            pl.BlockSpec((1, NCL), lambda j: (0, 0)),
            pl.BlockSpec((NCL, D), lambda j: (0, 0)),
            pl.BlockSpec((D, D), lambda j: (0, 0)),
            pl.BlockSpec((1, D), lambda j: (0, 0)),
            pl.BlockSpec((1, D), lambda j: (0, 0)),
            pl.BlockSpec((1, D), lambda j: (0, 0)),
        ],
        out_specs=pl.BlockSpec((RB, D), lambda j: (j, 0)),
        out_shape=jax.ShapeDtypeStruct((N, D), jnp.float32),
    )(acc2, zt, dinv, batch2, smile3, wproj, bproj2, wrep, wh, bconv, gsc,
      bsh)


def _final_body(acc2_ref, zt_ref, dinv_ref, batch_ref, bconv_ref, gsc_ref,
                bsh_ref, lin1_ref, b1_ref, lin2_ref, b2_ref, out_ref,
                sums_ref, cnts_ref):
    j = pl.program_id(0)
    h = _post(acc2_ref, zt_ref, dinv_ref, bconv_ref, gsc_ref, bsh_ref)
    oh = (batch_ref[...] ==
          lax.broadcasted_iota(jnp.int32, (RB, G), 1)).astype(jnp.float32)
    contrib = lax.dot_general(oh, h, (((0,), (0,)), ((), ())),
                              preferred_element_type=jnp.float32)
    ccnt = lax.dot_general(oh, jnp.ones((RB, D), jnp.float32),
                           (((0,), (0,)), ((), ())),
                           preferred_element_type=jnp.float32)

    @pl.when(j == 0)
    def _():
        sums_ref[...] = contrib
        cnts_ref[...] = ccnt

    @pl.when(j > 0)
    def _():
        sums_ref[...] = sums_ref[...] + contrib
        cnts_ref[...] = cnts_ref[...] + ccnt

    @pl.when(j == NRB - 1)
    def _():
        pooled = sums_ref[...] / jnp.maximum(cnts_ref[...], 1.0)
        o = jnp.maximum(
            lax.dot_general(pooled, lin1_ref[...], (((1,), (0,)), ((), ())),
                            preferred_element_type=jnp.float32)
            + b1_ref[...], 0.0)
        out_ref[...] = lax.dot_general(o, lin2_ref[...],
                                       (((1,), (0,)), ((), ())),
                                       preferred_element_type=jnp.float32) \
            + b2_ref[...]


def _tc_final(acc2, zt, dinv, batch2, bconv, gsc, bsh, lin1, b1, lin2, b2):
    return pl.pallas_call(
        _final_body,
        grid=(NRB,),
        in_specs=[
            pl.BlockSpec((NC, RB, D), lambda j: (0, j, 0)),
            pl.BlockSpec((RB, D), lambda j: (j, 0)),
            pl.BlockSpec((RB, 1), lambda j: (j, 0)),
            pl.BlockSpec((RB, 1), lambda j: (j, 0)),
            pl.BlockSpec((1, D), lambda j: (0, 0)),
            pl.BlockSpec((1, D), lambda j: (0, 0)),
            pl.BlockSpec((1, D), lambda j: (0, 0)),
            pl.BlockSpec((D, D), lambda j: (0, 0)),
            pl.BlockSpec((1, D), lambda j: (0, 0)),
            pl.BlockSpec((D, NCL), lambda j: (0, 0)),
            pl.BlockSpec((1, NCL), lambda j: (0, 0)),
        ],
        out_specs=pl.BlockSpec((G, NCL), lambda j: (0, 0)),
        out_shape=jax.ShapeDtypeStruct((G, NCL), jnp.float32),
        scratch_shapes=[
            pltpu.VMEM((G, D), jnp.float32),
            pltpu.VMEM((G, D), jnp.float32),
        ],
    )(acc2, zt, dinv, batch2, bconv, gsc, bsh, lin1, b1, lin2, b2)


def kernel(x, edge_index, batch_indice, smile_llm, params):
    e4 = edge_index.astype(jnp.int32).reshape(2, NW, NCHUNK, CH)
    batch2 = batch_indice.astype(jnp.int32).reshape(N, 1)

    wproj = params['W_proj']
    bproj2 = params['b_proj'].reshape(1, NCL)
    lin2 = params['lin2_W']
    b2 = params['lin2_b'].reshape(1, NCL)
    b1 = params['lin1_b'].reshape(1, D)
    bn_scale = 1.0 / jnp.sqrt(jnp.float32(1.0 + EPS))

    degp = _deg_kernel(e4)

    c0 = params['convs'][0]
    dinv, zt = _tc_layer0(degp, x, batch2, smile_llm, wproj, bproj2,
                          c0['W'][D:], c0['W'][:D])

    for i in range(1, 6):
        acc2 = _mp_kernel(zt, e4)
        ci = params['convs'][i]
        bni = params['bns'][i - 1]
        cprev = params['convs'][i - 1]
        zt = _tc_mid(acc2, zt, dinv, batch2, smile_llm, wproj, bproj2,
                     ci['W'][D:], ci['W'][:D], cprev['b'].reshape(1, D),
                     (bni['g'] * bn_scale).reshape(1, D),
                     bni['b'].reshape(1, D))

    acc2 = _mp_kernel(zt, e4)
    c5 = params['convs'][5]
    bn5 = params['bns'][5]
    return _tc_final(acc2, zt, dinv, batch2, c5['b'].reshape(1, D),
                     (bn5['g'] * bn_scale).reshape(1, D),
                     bn5['b'].reshape(1, D), params['lin1_W'], b1, lin2, b2)
